# Initial kernel scaffold; baseline (speedup 1.0000x reference)
import jax
import jax.numpy as jnp
from jax.experimental import pallas as pl


def kernel(xs, c0, h0, wx, wh, b):
    raise NotImplementedError("write your pallas kernel here")


# speedup vs baseline: 1.0037x; 1.0037x over previous
import functools

import jax
import jax.numpy as jnp
from jax.experimental import pallas as pl
from jax.experimental.pallas import tpu as pltpu


def _round_up(x, m):
    return (x + m - 1) // m * m


def _sigm(z):
    return 0.5 * jnp.tanh(0.5 * z) + 0.5


def _seq_body(xs_ref, c0_ref, h0_ref, wx_ref, wh_ref, b_ref,
              hs_ref, c_fin_ref, c_s, h_s, *, hp):
    t = pl.program_id(0)

    @pl.when(t == 0)
    def _():
        c_s[...] = c0_ref[...]
        h_s[...] = h0_ref[...]

    x = xs_ref[0].astype(jnp.bfloat16)
    h = h_s[...].astype(jnp.bfloat16)

    gates = (jnp.dot(x, wx_ref[...], preferred_element_type=jnp.float32)
             + jnp.dot(h, wh_ref[...], preferred_element_type=jnp.float32)
             + b_ref[...])

    i_g = _sigm(gates[:, 0 * hp:1 * hp])
    f_g = _sigm(gates[:, 1 * hp:2 * hp])
    g_c = jnp.tanh(gates[:, 2 * hp:3 * hp])
    o_g = _sigm(gates[:, 3 * hp:4 * hp])

    c_new = f_g * c_s[...] + i_g * g_c
    h_new = o_g * jnp.tanh(c_new)
    c_s[...] = c_new
    h_s[...] = h_new
    hs_ref[0] = h_new

    @pl.when(t == pl.num_programs(0) - 1)
    def _():
        c_fin_ref[...] = c_new


def kernel(xs, c0, h0, wx, wh, b):
    T, B, D = xs.shape
    H = h0.shape[1]
    Hp4 = wx.shape[1]
    Hp = Hp4 // 4

    if Hp != H:
        c0 = jnp.pad(c0, ((0, 0), (0, Hp - H)))
        h0 = jnp.pad(h0, ((0, 0), (0, Hp - H)))

    wx_b = wx.astype(jnp.bfloat16)
    wh_b = wh.astype(jnp.bfloat16)
    b_f = b.astype(jnp.float32)

    body = functools.partial(_seq_body, hp=Hp)

    hs, c_fin = pl.pallas_call(
        body,
        out_shape=(
            jax.ShapeDtypeStruct((T, B, Hp), h0.dtype),
            jax.ShapeDtypeStruct((B, Hp), c0.dtype),
        ),
        grid=(T,),
        in_specs=[
            pl.BlockSpec((1, B, D), lambda t: (t, 0, 0)),
            pl.BlockSpec((B, Hp), lambda t: (0, 0)),
            pl.BlockSpec((B, Hp), lambda t: (0, 0)),
            pl.BlockSpec((D, Hp4), lambda t: (0, 0)),
            pl.BlockSpec((Hp, Hp4), lambda t: (0, 0)),
            pl.BlockSpec((1, Hp4), lambda t: (0, 0)),
        ],
        out_specs=(
            pl.BlockSpec((1, B, Hp), lambda t: (t, 0, 0)),
            pl.BlockSpec((B, Hp), lambda t: (0, 0)),
        ),
        scratch_shapes=[
            pltpu.VMEM((B, Hp), jnp.float32),
            pltpu.VMEM((B, Hp), jnp.float32),
        ],
        compiler_params=pltpu.CompilerParams(
            dimension_semantics=("arbitrary",),
            vmem_limit_bytes=48 * 1024 * 1024,
        ),
    )(xs, c0, h0, wx_b, wh_b, b_f)

    h_fin = hs[-1]
    if Hp != H:
        hs, c_fin, h_fin = hs[:, :, :H], c_fin[:, :H], h_fin[:, :H]
    return hs, c_fin, h_fin


# speedup vs baseline: 1.0442x; 1.0404x over previous
import functools

import jax
import jax.numpy as jnp
from jax.experimental import pallas as pl
from jax.experimental.pallas import tpu as pltpu


def _round_up(x, m):
    return (x + m - 1) // m * m


def _sigm(z):
    return 0.5 * jnp.tanh(0.5 * z) + 0.5


def _seq_body(xs_ref, c0_ref, h0_ref, wx_ref, wh_ref, b_ref,
              hs_ref, c_fin_ref, c_s, h_s, xg_s, *, hp, tt, bt):
    blk = pl.program_id(0)

    @pl.when(blk == 0)
    def _():
        c_s[...] = c0_ref[...]
        h_s[...] = h0_ref[...]

    xb = xs_ref[...].reshape(tt * bt, xs_ref.shape[2]).astype(jnp.bfloat16)
    xg_s[...] = (jnp.dot(xb, wx_ref[...], preferred_element_type=jnp.float32)
                 + b_ref[...])

    c = c_s[...]
    h = h_s[...]
    for k in range(tt):
        gates = (xg_s[k * bt:(k + 1) * bt, :]
                 + jnp.dot(h.astype(jnp.bfloat16), wh_ref[...],
                           preferred_element_type=jnp.float32))
        i_g = _sigm(gates[:, 0 * hp:1 * hp])
        f_g = _sigm(gates[:, 1 * hp:2 * hp])
        g_c = jnp.tanh(gates[:, 2 * hp:3 * hp])
        o_g = _sigm(gates[:, 3 * hp:4 * hp])
        c = f_g * c + i_g * g_c
        h = o_g * jnp.tanh(c)
        hs_ref[k] = h

    c_s[...] = c
    h_s[...] = h

    @pl.when(blk == pl.num_programs(0) - 1)
    def _():
        c_fin_ref[...] = c


def kernel(xs, c0, h0, wx, wh, b):
    T, B, D = xs.shape
    H = h0.shape[1]
    Hp4 = wx.shape[1]
    Hp = Hp4 // 4

    if Hp != H:
        c0 = jnp.pad(c0, ((0, 0), (0, Hp - H)))
        h0 = jnp.pad(h0, ((0, 0), (0, Hp - H)))

    wx_b = wx.astype(jnp.bfloat16)
    wh_b = wh.astype(jnp.bfloat16)
    b_f = b.astype(jnp.float32)

    tt = 8
    while T % tt:
        tt //= 2

    body = functools.partial(_seq_body, hp=Hp, tt=tt, bt=B)

    hs, c_fin = pl.pallas_call(
        body,
        out_shape=(
            jax.ShapeDtypeStruct((T, B, Hp), h0.dtype),
            jax.ShapeDtypeStruct((B, Hp), c0.dtype),
        ),
        grid=(T // tt,),
        in_specs=[
            pl.BlockSpec((tt, B, D), lambda i: (i, 0, 0)),
            pl.BlockSpec((B, Hp), lambda i: (0, 0)),
            pl.BlockSpec((B, Hp), lambda i: (0, 0)),
            pl.BlockSpec((D, Hp4), lambda i: (0, 0)),
            pl.BlockSpec((Hp, Hp4), lambda i: (0, 0)),
            pl.BlockSpec((1, Hp4), lambda i: (0, 0)),
        ],
        out_specs=(
            pl.BlockSpec((tt, B, Hp), lambda i: (i, 0, 0)),
            pl.BlockSpec((B, Hp), lambda i: (0, 0)),
        ),
        scratch_shapes=[
            pltpu.VMEM((B, Hp), jnp.float32),
            pltpu.VMEM((B, Hp), jnp.float32),
            pltpu.VMEM((tt * B, Hp4), jnp.float32),
        ],
        compiler_params=pltpu.CompilerParams(
            dimension_semantics=("arbitrary",),
            vmem_limit_bytes=56 * 1024 * 1024,
        ),
    )(xs, c0, h0, wx_b, wh_b, b_f)

    h_fin = hs[-1]
    if Hp != H:
        hs, c_fin, h_fin = hs[:, :, :H], c_fin[:, :H], h_fin[:, :H]
    return hs, c_fin, h_fin


# speedup vs baseline: 1.0771x; 1.0314x over previous
import functools

import jax
import jax.numpy as jnp
from jax.experimental import pallas as pl
from jax.experimental.pallas import tpu as pltpu


def _round_up(x, m):
    return (x + m - 1) // m * m


def _sigm(z):
    return 0.5 * jnp.tanh(0.5 * z) + 0.5


def _seq_body(xs_ref, c0_ref, h0_ref, wx_ref, wh_ref, b_ref,
              hs_ref, c_fin_ref, c_s, h_s, *, hp, tt):
    blk = pl.program_id(0)

    @pl.when(blk == 0)
    def _():
        c_s[...] = c0_ref[...]
        h_s[...] = h0_ref[...]

    c = c_s[...]
    h = h_s[...]
    for k in range(tt):
        x = xs_ref[k].astype(jnp.bfloat16)
        gates = (jnp.dot(x, wx_ref[...], preferred_element_type=jnp.float32)
                 + jnp.dot(h.astype(jnp.bfloat16), wh_ref[...],
                           preferred_element_type=jnp.float32)
                 + b_ref[...])
        i_g = _sigm(gates[:, 0 * hp:1 * hp])
        f_g = _sigm(gates[:, 1 * hp:2 * hp])
        g_c = jnp.tanh(gates[:, 2 * hp:3 * hp])
        o_g = _sigm(gates[:, 3 * hp:4 * hp])
        c = f_g * c + i_g * g_c
        h = o_g * jnp.tanh(c)
        hs_ref[k] = h

    c_s[...] = c
    h_s[...] = h

    @pl.when(blk == pl.num_programs(0) - 1)
    def _():
        c_fin_ref[...] = c


def kernel(xs, c0, h0, wx, wh, b):
    T, B, D = xs.shape
    H = h0.shape[1]
    Hp4 = wx.shape[1]
    Hp = Hp4 // 4

    if Hp != H:
        c0 = jnp.pad(c0, ((0, 0), (0, Hp - H)))
        h0 = jnp.pad(h0, ((0, 0), (0, Hp - H)))

    wx_b = wx.astype(jnp.bfloat16)
    wh_b = wh.astype(jnp.bfloat16)
    b_f = b.astype(jnp.float32)

    tt = 8
    while T % tt:
        tt //= 2

    body = functools.partial(_seq_body, hp=Hp, tt=tt)

    hs, c_fin = pl.pallas_call(
        body,
        out_shape=(
            jax.ShapeDtypeStruct((T, B, Hp), h0.dtype),
            jax.ShapeDtypeStruct((B, Hp), c0.dtype),
        ),
        grid=(T // tt,),
        in_specs=[
            pl.BlockSpec((tt, B, D), lambda i: (i, 0, 0)),
            pl.BlockSpec((B, Hp), lambda i: (0, 0)),
            pl.BlockSpec((B, Hp), lambda i: (0, 0)),
            pl.BlockSpec((D, Hp4), lambda i: (0, 0)),
            pl.BlockSpec((Hp, Hp4), lambda i: (0, 0)),
            pl.BlockSpec((1, Hp4), lambda i: (0, 0)),
        ],
        out_specs=(
            pl.BlockSpec((tt, B, Hp), lambda i: (i, 0, 0)),
            pl.BlockSpec((B, Hp), lambda i: (0, 0)),
        ),
        scratch_shapes=[
            pltpu.VMEM((B, Hp), jnp.float32),
            pltpu.VMEM((B, Hp), jnp.float32),
        ],
        compiler_params=pltpu.CompilerParams(
            dimension_semantics=("arbitrary",),
            vmem_limit_bytes=56 * 1024 * 1024,
        ),
    )(xs, c0, h0, wx_b, wh_b, b_f)

    h_fin = hs[-1]
    if Hp != H:
        hs, c_fin, h_fin = hs[:, :, :H], c_fin[:, :H], h_fin[:, :H]
    return hs, c_fin, h_fin


# speedup vs baseline: 1.0791x; 1.0019x over previous
import functools

import jax
import jax.numpy as jnp
from jax.experimental import pallas as pl
from jax.experimental.pallas import tpu as pltpu


def _round_up(x, m):
    return (x + m - 1) // m * m


def _sigm(z):
    return 0.5 * jnp.tanh(0.5 * z) + 0.5


def _seq_body(xs_ref, c0_ref, h0_ref, wx_ref, wh_ref, b_ref,
              hs_ref, c_fin_ref, c_s, h_s, *, hp, tt):
    blk = pl.program_id(0)

    @pl.when(blk == 0)
    def _():
        c_s[...] = c0_ref[...]
        h_s[...] = h0_ref[...]

    c = c_s[...]
    h = h_s[...]
    for k in range(tt):
        x = xs_ref[k].astype(jnp.bfloat16)
        gates = (jnp.dot(x, wx_ref[...], preferred_element_type=jnp.float32)
                 + jnp.dot(h.astype(jnp.bfloat16), wh_ref[...],
                           preferred_element_type=jnp.float32)
                 + b_ref[...])
        i_g = _sigm(gates[:, 0 * hp:1 * hp])
        f_g = _sigm(gates[:, 1 * hp:2 * hp])
        g_c = jnp.tanh(gates[:, 2 * hp:3 * hp])
        o_g = _sigm(gates[:, 3 * hp:4 * hp])
        c = f_g * c + i_g * g_c
        h = o_g * jnp.tanh(c)
        hs_ref[k] = h

    c_s[...] = c
    h_s[...] = h

    @pl.when(blk == pl.num_programs(0) - 1)
    def _():
        c_fin_ref[...] = c


def kernel(xs, c0, h0, wx, wh, b):
    T, B, D = xs.shape
    H = h0.shape[1]
    Hp4 = wx.shape[1]
    Hp = Hp4 // 4

    if Hp != H:
        c0 = jnp.pad(c0, ((0, 0), (0, Hp - H)))
        h0 = jnp.pad(h0, ((0, 0), (0, Hp - H)))

    wx_b = wx.astype(jnp.bfloat16)
    wh_b = wh.astype(jnp.bfloat16)
    b_f = b.astype(jnp.float32)

    tt = 16
    while T % tt:
        tt //= 2

    body = functools.partial(_seq_body, hp=Hp, tt=tt)

    hs, c_fin = pl.pallas_call(
        body,
        out_shape=(
            jax.ShapeDtypeStruct((T, B, Hp), h0.dtype),
            jax.ShapeDtypeStruct((B, Hp), c0.dtype),
        ),
        grid=(T // tt,),
        in_specs=[
            pl.BlockSpec((tt, B, D), lambda i: (i, 0, 0)),
            pl.BlockSpec((B, Hp), lambda i: (0, 0)),
            pl.BlockSpec((B, Hp), lambda i: (0, 0)),
            pl.BlockSpec((D, Hp4), lambda i: (0, 0)),
            pl.BlockSpec((Hp, Hp4), lambda i: (0, 0)),
            pl.BlockSpec((1, Hp4), lambda i: (0, 0)),
        ],
        out_specs=(
            pl.BlockSpec((tt, B, Hp), lambda i: (i, 0, 0)),
            pl.BlockSpec((B, Hp), lambda i: (0, 0)),
        ),
        scratch_shapes=[
            pltpu.VMEM((B, Hp), jnp.float32),
            pltpu.VMEM((B, Hp), jnp.float32),
        ],
        compiler_params=pltpu.CompilerParams(
            dimension_semantics=("arbitrary",),
            vmem_limit_bytes=56 * 1024 * 1024,
        ),
    )(xs, c0, h0, wx_b, wh_b, b_f)

    h_fin = hs[-1]
    if Hp != H:
        hs, c_fin, h_fin = hs[:, :, :H], c_fin[:, :H], h_fin[:, :H]
    return hs, c_fin, h_fin


# speedup vs baseline: 1.1311x; 1.0482x over previous
import functools

import jax
import jax.numpy as jnp
from jax.experimental import pallas as pl
from jax.experimental.pallas import tpu as pltpu


def _round_up(x, m):
    return (x + m - 1) // m * m


def _sigm(z):
    return 0.5 * jnp.tanh(0.5 * z) + 0.5


def _seq_body(xs_ref, c0_ref, h0_ref, wx_ref, wh_ref, b_ref,
              hs_ref, c_fin_ref, c_s, h_s, *, hp, tt):
    blk = pl.program_id(0)

    @pl.when(blk == 0)
    def _():
        c_s[...] = c0_ref[...]
        h_s[...] = h0_ref[...]

    bt = c_s.shape[0]
    nh = 2 if bt % 256 == 0 else 1
    bh = bt // nh

    cs = [c_s[j * bh:(j + 1) * bh, :] for j in range(nh)]
    hs = [h_s[j * bh:(j + 1) * bh, :] for j in range(nh)]
    for k in range(tt):
        for j in range(nh):
            x = xs_ref[k, j * bh:(j + 1) * bh, :].astype(jnp.bfloat16)
            gates = (jnp.dot(x, wx_ref[...],
                             preferred_element_type=jnp.float32)
                     + jnp.dot(hs[j].astype(jnp.bfloat16), wh_ref[...],
                               preferred_element_type=jnp.float32)
                     + b_ref[...])
            i_g = _sigm(gates[:, 0 * hp:1 * hp])
            f_g = _sigm(gates[:, 1 * hp:2 * hp])
            g_c = jnp.tanh(gates[:, 2 * hp:3 * hp])
            o_g = _sigm(gates[:, 3 * hp:4 * hp])
            cs[j] = f_g * cs[j] + i_g * g_c
            hs[j] = o_g * jnp.tanh(cs[j])
            hs_ref[k, j * bh:(j + 1) * bh, :] = hs[j]

    for j in range(nh):
        c_s[j * bh:(j + 1) * bh, :] = cs[j]
        h_s[j * bh:(j + 1) * bh, :] = hs[j]

    @pl.when(blk == pl.num_programs(0) - 1)
    def _():
        for j in range(nh):
            c_fin_ref[j * bh:(j + 1) * bh, :] = cs[j]


def kernel(xs, c0, h0, wx, wh, b):
    T, B, D = xs.shape
    H = h0.shape[1]
    Hp4 = wx.shape[1]
    Hp = Hp4 // 4

    if Hp != H:
        c0 = jnp.pad(c0, ((0, 0), (0, Hp - H)))
        h0 = jnp.pad(h0, ((0, 0), (0, Hp - H)))

    wx_b = wx.astype(jnp.bfloat16)
    wh_b = wh.astype(jnp.bfloat16)
    b_f = b.astype(jnp.float32)

    tt = 8
    while T % tt:
        tt //= 2

    body = functools.partial(_seq_body, hp=Hp, tt=tt)

    hs, c_fin = pl.pallas_call(
        body,
        out_shape=(
            jax.ShapeDtypeStruct((T, B, Hp), h0.dtype),
            jax.ShapeDtypeStruct((B, Hp), c0.dtype),
        ),
        grid=(T // tt,),
        in_specs=[
            pl.BlockSpec((tt, B, D), lambda i: (i, 0, 0)),
            pl.BlockSpec((B, Hp), lambda i: (0, 0)),
            pl.BlockSpec((B, Hp), lambda i: (0, 0)),
            pl.BlockSpec((D, Hp4), lambda i: (0, 0)),
            pl.BlockSpec((Hp, Hp4), lambda i: (0, 0)),
            pl.BlockSpec((1, Hp4), lambda i: (0, 0)),
        ],
        out_specs=(
            pl.BlockSpec((tt, B, Hp), lambda i: (i, 0, 0)),
            pl.BlockSpec((B, Hp), lambda i: (0, 0)),
        ),
        scratch_shapes=[
            pltpu.VMEM((B, Hp), jnp.float32),
            pltpu.VMEM((B, Hp), jnp.float32),
        ],
        compiler_params=pltpu.CompilerParams(
            dimension_semantics=("arbitrary",),
            vmem_limit_bytes=56 * 1024 * 1024,
        ),
    )(xs, c0, h0, wx_b, wh_b, b_f)

    h_fin = hs[-1]
    if Hp != H:
        hs, c_fin, h_fin = hs[:, :, :H], c_fin[:, :H], h_fin[:, :H]
    return hs, c_fin, h_fin
